# Initial kernel scaffold; baseline (speedup 1.0000x reference)
import jax
import jax.numpy as jnp
from jax.experimental import pallas as pl


def kernel(x, edge_index, W1, b1, W2, b2):
    raise NotImplementedError("write your pallas kernel here")


# speedup vs baseline: 9.9456x; 9.9456x over previous
import jax
import jax.numpy as jnp
from jax import lax
from jax.experimental import pallas as pl
from jax.experimental.pallas import tpu as pltpu
from jax.experimental.pallas import tpu_sc as plsc

N_NODES = 10000
N_EDGES = 320000
D = 128
NC = 2
NS = 16
NW = NC * NS
CHUNK = 128
CHUNKS_PER_W = -(-(N_EDGES // NW) // CHUNK)
EPW = CHUNKS_PER_W * CHUNK
EP = EPW * NW
NP = 10112
RPT = NP // NS
HW = 128
RB = 1264

_MESH = plsc.VectorSubcoreMesh(core_axis_name="c", subcore_axis_name="s")


def _sc_hist_body(dst_hbm, zeros_hbm, ones_hbm, hist_out, hist_sh, ones_v, idx_v):
    c = lax.axis_index("c")
    s = lax.axis_index("s")
    wid = s * NC + c
    row0 = pl.multiple_of(s * RPT, 8)
    pltpu.sync_copy(zeros_hbm.at[pl.ds(row0, RPT)], hist_sh.at[pl.ds(row0, RPT)])
    pltpu.sync_copy(ones_hbm, ones_v)
    plsc.subcore_barrier()
    base = wid * EPW

    def body(j, carry):
        off = pl.multiple_of(base + j * CHUNK, CHUNK)
        pltpu.sync_copy(dst_hbm.at[pl.ds(off, CHUNK)], idx_v)
        pltpu.sync_copy(ones_v, hist_sh.at[idx_v], add=True)
        return carry

    lax.fori_loop(0, CHUNKS_PER_W, body, 0)
    plsc.subcore_barrier()
    pltpu.sync_copy(hist_sh.at[pl.ds(row0, RPT)], hist_out.at[c, pl.ds(row0, RPT)])


_sc_hist = pl.kernel(
    _sc_hist_body,
    out_type=jax.ShapeDtypeStruct((NC, NP, HW), jnp.float32),
    mesh=_MESH,
    scratch_types=[
        pltpu.VMEM_SHARED((NP, HW), jnp.float32),
        pltpu.VMEM((CHUNK, HW), jnp.float32),
        pltpu.VMEM((CHUNK,), jnp.int32),
    ],
)


def _sc_scatter_body(hs_hbm, src_hbm, dst_hbm, zeros_hbm, part_out,
                     acc_sh, rows_v, sidx_v, didx_v, sem):
    c = lax.axis_index("c")
    s = lax.axis_index("s")
    wid = s * NC + c
    row0 = pl.multiple_of(s * RPT, 8)
    pltpu.sync_copy(zeros_hbm.at[pl.ds(row0, RPT)], acc_sh.at[pl.ds(row0, RPT)])
    plsc.subcore_barrier()
    base = wid * EPW

    def body(j, carry):
        off = pl.multiple_of(base + j * CHUNK, CHUNK)
        pltpu.sync_copy(src_hbm.at[pl.ds(off, CHUNK)], sidx_v)
        pltpu.sync_copy(dst_hbm.at[pl.ds(off, CHUNK)], didx_v)
        pltpu.async_copy(hs_hbm.at[sidx_v], rows_v, sem).wait()
        pltpu.sync_copy(rows_v, acc_sh.at[didx_v], add=True)
        return carry

    lax.fori_loop(0, CHUNKS_PER_W, body, 0)
    plsc.subcore_barrier()
    pltpu.sync_copy(acc_sh.at[pl.ds(row0, RPT)], part_out.at[c, pl.ds(row0, RPT)])


_sc_scatter = pl.kernel(
    _sc_scatter_body,
    out_type=jax.ShapeDtypeStruct((NC, NP, D), jnp.float32),
    mesh=_MESH,
    scratch_types=[
        pltpu.VMEM_SHARED((NP, D), jnp.float32),
        pltpu.VMEM((CHUNK, D), jnp.float32),
        pltpu.VMEM((CHUNK,), jnp.int32),
        pltpu.VMEM((CHUNK,), jnp.int32),
        pltpu.SemaphoreType.DMA,
    ],
)


def _dis(hist_ref):
    deg = hist_ref[0, :, 0:1] + hist_ref[1, :, 0:1] + 1.0
    return lax.rsqrt(deg)


def _tc_first_body(x_ref, w_ref, hist_ref, hs_ref):
    dis = _dis(hist_ref)
    hs_ref[...] = dis * jnp.dot(x_ref[...], w_ref[...],
                                preferred_element_type=jnp.float32)


_tc_first = pl.pallas_call(
    _tc_first_body,
    grid=(NP // RB,),
    in_specs=[
        pl.BlockSpec((RB, D), lambda i: (i, 0)),
        pl.BlockSpec((D, D), lambda i: (0, 0)),
        pl.BlockSpec((NC, RB, HW), lambda i: (0, i, 0)),
    ],
    out_specs=pl.BlockSpec((RB, D), lambda i: (i, 0)),
    out_shape=jax.ShapeDtypeStruct((NP, D), jnp.float32),
)


def _tc_mid_body(hist_ref, p_ref, hs_ref, w_ref, b_ref, out_ref):
    dis = _dis(hist_ref)
    acc = p_ref[0] + p_ref[1] + hs_ref[...]
    h2 = jnp.maximum(dis * acc + b_ref[...], 0.0)
    out_ref[...] = dis * jnp.dot(h2, w_ref[...],
                                 preferred_element_type=jnp.float32)


_tc_mid = pl.pallas_call(
    _tc_mid_body,
    grid=(NP // RB,),
    in_specs=[
        pl.BlockSpec((NC, RB, HW), lambda i: (0, i, 0)),
        pl.BlockSpec((NC, RB, D), lambda i: (0, i, 0)),
        pl.BlockSpec((RB, D), lambda i: (i, 0)),
        pl.BlockSpec((D, D), lambda i: (0, 0)),
        pl.BlockSpec((1, D), lambda i: (0, 0)),
    ],
    out_specs=pl.BlockSpec((RB, D), lambda i: (i, 0)),
    out_shape=jax.ShapeDtypeStruct((NP, D), jnp.float32),
)


def _tc_last_body(hist_ref, p_ref, hs_ref, b_ref, out_ref):
    dis = _dis(hist_ref)
    acc = p_ref[0] + p_ref[1] + hs_ref[...]
    out_ref[...] = dis * acc + b_ref[...]


_tc_last = pl.pallas_call(
    _tc_last_body,
    grid=(NP // RB,),
    in_specs=[
        pl.BlockSpec((NC, RB, HW), lambda i: (0, i, 0)),
        pl.BlockSpec((NC, RB, D), lambda i: (0, i, 0)),
        pl.BlockSpec((RB, D), lambda i: (i, 0)),
        pl.BlockSpec((1, D), lambda i: (0, 0)),
    ],
    out_specs=pl.BlockSpec((RB, D), lambda i: (i, 0)),
    out_shape=jax.ShapeDtypeStruct((NP, D), jnp.float32),
)


def kernel(x, edge_index, W1, b1, W2, b2):
    pad_e = jnp.full((EP - N_EDGES,), N_NODES, jnp.int32)
    srcp = jnp.concatenate([edge_index[0], pad_e])
    dstp = jnp.concatenate([edge_index[1], pad_e])
    xp = jnp.pad(x, ((0, NP - N_NODES), (0, 0)))
    zeros_nd = jnp.zeros((NP, D), jnp.float32)
    ones_ch = jnp.ones((CHUNK, HW), jnp.float32)

    hist = _sc_hist(dstp, zeros_nd, ones_ch)
    hs1 = _tc_first(xp, W1, hist)
    p = _sc_scatter(hs1, srcp, dstp, zeros_nd)
    hs2 = _tc_mid(hist, p, hs1, W2, b1.reshape(1, D))
    q = _sc_scatter(hs2, srcp, dstp, zeros_nd)
    out = _tc_last(hist, q, hs2, b2.reshape(1, D))
    return out[:N_NODES]
